# baseline (speedup 1.0000x reference)
import functools

import jax
import jax.numpy as jnp
from jax import lax
from jax.experimental import pallas as pl
from jax.experimental.pallas import tpu as pltpu
from jax.experimental.pallas import tpu_sc as plsc

NPTS = 2_000_000
NB = 1_000_000
NBODY = 999_936
NBLK = 7_812
NTAIL = NB - NBODY
BODY_B = NBLK * 512
NW = 32
NSID = 16
BLK_BIG = 489
BLK_SML = 488
CHUNK_BLK = 32
NCH = 15
CELLS = 8_880
CELLS_PAD = 9_216
PER_W_CELLS = CELLS_PAD // NW

_MESH = plsc.VectorSubcoreMesh(core_axis_name="c", subcore_axis_name="s")
_CPARAMS = pltpu.CompilerParams(needs_layout_passes=False)


def _wid():
    return lax.axis_index("s") * 2 + lax.axis_index("c")


def _round_bf16(x):
    u = plsc.bitcast(x, jnp.int32)
    u = (u + 0x7FFF + ((u >> 16) & 1)) & ~0xFFFF
    return plsc.bitcast(u, jnp.float32)


def _bcast12(tv):
    tvec = _round_bf16(tv[...])
    return [jnp.full((16,), tvec[k], jnp.float32) for k in range(12)]


@functools.partial(
    pl.kernel,
    out_type=jax.ShapeDtypeStruct((NW * CELLS_PAD,), jnp.int32),
    mesh=_MESH,
    compiler_params=_CPARAMS,
    scratch_types=[
        pltpu.VMEM((CHUNK_BLK * 512,), jnp.float32),
        pltpu.VMEM((CELLS_PAD,), jnp.int32),
        pltpu.VMEM((16,), jnp.int32),
        pltpu.VMEM((16,), jnp.int32),
        pltpu.VMEM((16,), jnp.int32),
        pltpu.VMEM((16,), jnp.int32),
        pltpu.VMEM((16,), jnp.float32),
    ],
)
def _phase_a(body_hbm, tail_hbm, trans_hbm, out_hbm, buf, bestn,
             kbuf, abuf, cbuf, dbuf, tv):
    wid = _wid()
    b = wid % 2
    sid = wid // 2
    pltpu.sync_copy(trans_hbm, tv)
    t = _bcast12(tv)
    iota = lax.iota(jnp.int32, 16)
    minus1 = jnp.full((16,), -1, jnp.int32)

    def init(i, _):
        bestn[pl.ds(i * 16, 16)] = minus1
        return 0
    lax.fori_loop(0, CELLS_PAD // 16, init, 0)

    badd = b * 4440
    iota_n = jnp.minimum(iota + 1, 15)

    def point_vreg(i0, i1, i2, i3, po, nbase, spill):
        X = t[0] * i0 + t[1] * i1 + t[2] * i2 + t[3] * i3
        Y = t[4] * i0 + t[5] * i1 + t[6] * i2 + t[7] * i3
        Z = t[8] * i0 + t[9] * i1 + t[10] * i2 + t[11] * i3
        xi = jnp.minimum(jnp.maximum(X / Z, 0.0), 36.0).astype(jnp.int32)
        yi = jnp.minimum(jnp.maximum(Y / Z, 0.0), 119.0).astype(jnp.int32)
        cell = xi * 120 + yi + badd
        cell = jnp.where(Z > 0.0, cell, CELLS_PAD - 1)
        sk = jnp.sort(cell * 4096 + po)
        spill[...] = sk
        knext = plsc.load_gather(spill, [iota_n])
        last = ((sk >> 12) != (knext >> 12)) | (iota == 15)
        plsc.store_scatter(bestn, [sk >> 12], nbase + (sk & 4095), mask=last)

    blk_base = sid * BLK_SML + jnp.minimum(sid, 4)

    def do_chunk(blk0, nblk):
        pltpu.sync_copy(body_hbm.at[pl.ds(b * BODY_B + blk0 * 512, nblk * 512)],
                        buf.at[pl.ds(0, nblk * 512)])
        nbase = b * NB + blk0 * 128

        def one(v, spill):
            k = v // 8
            w = (v % 8) * 16
            o = k * 512 + w
            i0 = _round_bf16(buf[pl.ds(o, 16)])
            i1 = _round_bf16(buf[pl.ds(o + 128, 16)])
            i2 = _round_bf16(buf[pl.ds(o + 256, 16)])
            i3 = _round_bf16(buf[pl.ds(o + 384, 16)])
            point_vreg(i0, i1, i2, i3, k * 128 + w + iota, nbase, spill)

        def body_fn(u, _):
            one(u * 4, kbuf)
            one(u * 4 + 1, abuf)
            one(u * 4 + 2, cbuf)
            one(u * 4 + 3, dbuf)
            return 0
        lax.fori_loop(0, nblk * 2, body_fn, 0)

    for ci in range(NCH):
        do_chunk(blk_base + ci * CHUNK_BLK, CHUNK_BLK)

    @pl.when(sid < 4)
    def _tail_big():
        do_chunk(blk_base + NCH * CHUNK_BLK, BLK_BIG - NCH * CHUNK_BLK)

    @pl.when(sid >= 4)
    def _tail_small():
        do_chunk(blk_base + NCH * CHUNK_BLK, BLK_SML - NCH * CHUNK_BLK)

    @pl.when(sid == NSID - 1)
    def _ragged():
        pltpu.sync_copy(tail_hbm.at[pl.ds(b * 256, 256)], buf.at[pl.ds(0, 256)])
        for v in range(4):
            w = v * 16
            i0 = _round_bf16(buf[pl.ds(w, 16)])
            i1 = _round_bf16(buf[pl.ds(w + 64, 16)])
            i2 = _round_bf16(buf[pl.ds(w + 128, 16)])
            i3 = _round_bf16(buf[pl.ds(w + 192, 16)])
            point_vreg(i0, i1, i2, i3, w + iota, b * NB + NBODY, kbuf)

    pltpu.sync_copy(bestn, out_hbm.at[pl.ds(wid * CELLS_PAD, CELLS_PAD)])


@functools.partial(
    pl.kernel,
    out_type=jax.ShapeDtypeStruct((CELLS_PAD,), jnp.float32),
    mesh=_MESH,
    compiler_params=_CPARAMS,
    scratch_types=[
        pltpu.VMEM((NW * PER_W_CELLS,), jnp.int32),
        pltpu.VMEM((PER_W_CELLS,), jnp.int32),
        pltpu.VMEM((PER_W_CELLS * 4,), jnp.int32),
        pltpu.VMEM((PER_W_CELLS * 4,), jnp.float32),
        pltpu.VMEM((PER_W_CELLS,), jnp.float32),
        pltpu.VMEM((512,), jnp.float32),
        pltpu.VMEM((16,), jnp.float32),
        pltpu.SemaphoreType.DMA,
    ],
)
def _phase_b(body_hbm, tail_hbm, trans_hbm, bestn_hbm, out_hbm,
             loc, bestbuf, idxbuf, rows, outbuf, tailv, tv, sem):
    wid = _wid()
    cell0 = wid * PER_W_CELLS
    pltpu.sync_copy(trans_hbm, tv)
    pltpu.sync_copy(tail_hbm, tailv)
    t = _bcast12(tv)
    iota = lax.iota(jnp.int32, 16)
    iota4 = iota * 4
    for j in range(NW):
        pltpu.sync_copy(
            bestn_hbm.at[pl.ds(j * CELLS_PAD + cell0, PER_W_CELLS)],
            loc.at[pl.ds(j * PER_W_CELLS, PER_W_CELLS)])
    for v in range(PER_W_CELLS // 16):
        best = loc[pl.ds(v * 16, 16)]
        for j in range(1, NW):
            best = jnp.maximum(best, loc[pl.ds(j * PER_W_CELLS + v * 16, 16)])
        bestbuf[pl.ds(v * 16, 16)] = best
        cellv = cell0 + v * 16 + iota
        bsel = (best >= NB).astype(jnp.int32)
        n_l = best - bsel * NB
        body_ok = (best >= 0) & (n_l < NBODY)
        base_i = jnp.where(body_ok,
                           bsel * BODY_B + (n_l >> 7) * 512 + (n_l & 127),
                           cellv)
        for j in range(4):
            plsc.store_scatter(idxbuf, [v * 64 + iota4 + j], base_i + j * 128)
    pltpu.async_copy(body_hbm.at[idxbuf], rows, sem).wait()
    for v in range(PER_W_CELLS // 16):
        fbase = v * 64
        i0 = _round_bf16(plsc.load_gather(rows, [fbase + iota4]))
        i1 = _round_bf16(plsc.load_gather(rows, [fbase + iota4 + 1]))
        i2 = _round_bf16(plsc.load_gather(rows, [fbase + iota4 + 2]))
        i3 = _round_bf16(plsc.load_gather(rows, [fbase + iota4 + 3]))
        Zb = t[8] * i0 + t[9] * i1 + t[10] * i2 + t[11] * i3
        best = bestbuf[pl.ds(v * 16, 16)]
        bsel = (best >= NB).astype(jnp.int32)
        n_l = best - bsel * NB
        is_tail = (best >= 0) & (n_l >= NBODY)
        tix = jnp.minimum(jnp.maximum(n_l - NBODY, 0), NTAIL - 1) + bsel * 256
        j0 = _round_bf16(plsc.load_gather(tailv, [tix]))
        j1 = _round_bf16(plsc.load_gather(tailv, [tix + 64]))
        j2 = _round_bf16(plsc.load_gather(tailv, [tix + 128]))
        j3 = _round_bf16(plsc.load_gather(tailv, [tix + 192]))
        Zt = t[8] * j0 + t[9] * j1 + t[10] * j2 + t[11] * j3
        Z = jnp.where(is_tail, Zt, Zb)
        outbuf[pl.ds(v * 16, 16)] = jnp.where(best >= 0, Z, 0.0)
    pltpu.sync_copy(outbuf, out_hbm.at[pl.ds(cell0, PER_W_CELLS)])


def kernel(inputs, trans):
    body = lax.reshape(
        jnp.reshape(inputs[:, :NBODY, :], (2, NBLK, 128, 4)),
        (2 * BODY_B,), dimensions=(0, 1, 3, 2))
    tail = lax.reshape(inputs[:, NBODY:, :], (512,), dimensions=(0, 2, 1))
    tpad = jnp.zeros((16,), jnp.float32).at[:12].set(trans.ravel())
    bestn = _phase_a(body, tail, tpad)
    depth = _phase_b(body, tail, tpad, bestn)
    return depth[:CELLS].reshape(2, 37, 120)


# speedup vs baseline: 1.0072x; 1.0072x over previous
import functools

import jax
import jax.numpy as jnp
from jax import lax
from jax.experimental import pallas as pl
from jax.experimental.pallas import tpu as pltpu
from jax.experimental.pallas import tpu_sc as plsc

NPTS = 2_000_000
NB = 1_000_000
NBODY = 999_936
NBLK = 7_812
NTAIL = NB - NBODY
BODY_B = NBLK * 512
NW = 32
NSID = 16
BLK_BIG = 489
BLK_SML = 488
CHUNK_BLK = 32
NCH = 15
CELLS = 8_880
CELLS_PAD = 9_216
PER_W_CELLS = CELLS_PAD // NW

_MESH = plsc.VectorSubcoreMesh(core_axis_name="c", subcore_axis_name="s")
_CPARAMS = pltpu.CompilerParams(needs_layout_passes=False)


def _wid():
    return lax.axis_index("s") * 2 + lax.axis_index("c")


def _round_bf16(x):
    u = plsc.bitcast(x, jnp.int32)
    u = (u + 0x7FFF + ((u >> 16) & 1)) & ~0xFFFF
    return plsc.bitcast(u, jnp.float32)


def _bcast12(tv):
    tvec = _round_bf16(tv[...])
    return [jnp.full((16,), tvec[k], jnp.float32) for k in range(12)]


@functools.partial(
    pl.kernel,
    out_type=jax.ShapeDtypeStruct((NW * CELLS_PAD,), jnp.int32),
    mesh=_MESH,
    compiler_params=_CPARAMS,
    scratch_types=[
        pltpu.VMEM((CHUNK_BLK * 512,), jnp.float32),
        pltpu.VMEM((CELLS_PAD,), jnp.int32),
        pltpu.VMEM((16,), jnp.int32),
        pltpu.VMEM((16,), jnp.int32),
        pltpu.VMEM((16,), jnp.float32),
    ],
)
def _phase_a(body_hbm, tail_hbm, trans_hbm, out_hbm, buf, bestn, kbuf, abuf, tv):
    wid = _wid()
    b = wid % 2
    sid = wid // 2
    pltpu.sync_copy(trans_hbm, tv)
    t = _bcast12(tv)
    iota = lax.iota(jnp.int32, 16)
    minus1 = jnp.full((16,), -1, jnp.int32)

    def init(i, _):
        bestn[pl.ds(i * 16, 16)] = minus1
        return 0
    lax.fori_loop(0, CELLS_PAD // 16, init, 0)

    badd = b * 4440
    iota_n = jnp.minimum(iota + 1, 15)

    def point_vreg(i0, i1, i2, i3, po, nbase, spill):
        X = t[0] * i0 + t[1] * i1 + t[2] * i2 + t[3] * i3
        Y = t[4] * i0 + t[5] * i1 + t[6] * i2 + t[7] * i3
        Z = t[8] * i0 + t[9] * i1 + t[10] * i2 + t[11] * i3
        xi = jnp.minimum(jnp.maximum(X / Z, 0.0), 36.0).astype(jnp.int32)
        yi = jnp.minimum(jnp.maximum(Y / Z, 0.0), 119.0).astype(jnp.int32)
        cell = xi * 120 + yi + badd
        cell = jnp.where(Z > 0.0, cell, CELLS_PAD - 1)
        sk = jnp.sort(cell * 4096 + po)
        spill[...] = sk
        knext = plsc.load_gather(spill, [iota_n])
        last = ((sk >> 12) != (knext >> 12)) | (iota == 15)
        plsc.store_scatter(bestn, [sk >> 12], nbase + (sk & 4095), mask=last)

    blk_base = sid * BLK_SML + jnp.minimum(sid, 4)

    def do_chunk(blk0, nblk):
        pltpu.sync_copy(body_hbm.at[pl.ds(b * BODY_B + blk0 * 512, nblk * 512)],
                        buf.at[pl.ds(0, nblk * 512)])
        nbase = b * NB + blk0 * 128

        def one(v, spill):
            k = v // 8
            w = (v % 8) * 16
            o = k * 512 + w
            i0 = _round_bf16(buf[pl.ds(o, 16)])
            i1 = _round_bf16(buf[pl.ds(o + 128, 16)])
            i2 = _round_bf16(buf[pl.ds(o + 256, 16)])
            i3 = _round_bf16(buf[pl.ds(o + 384, 16)])
            point_vreg(i0, i1, i2, i3, k * 128 + w + iota, nbase, spill)

        def body_fn(u, _):
            one(u * 2, kbuf)
            one(u * 2 + 1, abuf)
            return 0
        lax.fori_loop(0, nblk * 4, body_fn, 0)

    for ci in range(NCH):
        do_chunk(blk_base + ci * CHUNK_BLK, CHUNK_BLK)

    @pl.when(sid < 4)
    def _tail_big():
        do_chunk(blk_base + NCH * CHUNK_BLK, BLK_BIG - NCH * CHUNK_BLK)

    @pl.when(sid >= 4)
    def _tail_small():
        do_chunk(blk_base + NCH * CHUNK_BLK, BLK_SML - NCH * CHUNK_BLK)

    @pl.when(sid == NSID - 1)
    def _ragged():
        pltpu.sync_copy(tail_hbm.at[pl.ds(b * 256, 256)], buf.at[pl.ds(0, 256)])
        for v in range(4):
            w = v * 16
            i0 = _round_bf16(buf[pl.ds(w, 16)])
            i1 = _round_bf16(buf[pl.ds(w + 64, 16)])
            i2 = _round_bf16(buf[pl.ds(w + 128, 16)])
            i3 = _round_bf16(buf[pl.ds(w + 192, 16)])
            point_vreg(i0, i1, i2, i3, w + iota, b * NB + NBODY, kbuf)

    pltpu.sync_copy(bestn, out_hbm.at[pl.ds(wid * CELLS_PAD, CELLS_PAD)])


@functools.partial(
    pl.kernel,
    out_type=jax.ShapeDtypeStruct((CELLS_PAD,), jnp.float32),
    mesh=_MESH,
    compiler_params=_CPARAMS,
    scratch_types=[
        pltpu.VMEM((NW * PER_W_CELLS,), jnp.int32),
        pltpu.VMEM((PER_W_CELLS,), jnp.int32),
        pltpu.VMEM((PER_W_CELLS * 4,), jnp.int32),
        pltpu.VMEM((PER_W_CELLS * 4,), jnp.float32),
        pltpu.VMEM((PER_W_CELLS,), jnp.float32),
        pltpu.VMEM((512,), jnp.float32),
        pltpu.VMEM((16,), jnp.float32),
        pltpu.SemaphoreType.DMA,
    ],
)
def _phase_b(body_hbm, tail_hbm, trans_hbm, bestn_hbm, out_hbm,
             loc, bestbuf, idxbuf, rows, outbuf, tailv, tv, sem):
    wid = _wid()
    cell0 = wid * PER_W_CELLS
    pltpu.sync_copy(trans_hbm, tv)
    pltpu.sync_copy(tail_hbm, tailv)
    t = _bcast12(tv)
    iota = lax.iota(jnp.int32, 16)
    iota4 = iota * 4
    for j in range(NW):
        pltpu.sync_copy(
            bestn_hbm.at[pl.ds(j * CELLS_PAD + cell0, PER_W_CELLS)],
            loc.at[pl.ds(j * PER_W_CELLS, PER_W_CELLS)])
    for v in range(PER_W_CELLS // 16):
        best = loc[pl.ds(v * 16, 16)]
        for j in range(1, NW):
            best = jnp.maximum(best, loc[pl.ds(j * PER_W_CELLS + v * 16, 16)])
        bestbuf[pl.ds(v * 16, 16)] = best
        cellv = cell0 + v * 16 + iota
        bsel = (best >= NB).astype(jnp.int32)
        n_l = best - bsel * NB
        body_ok = (best >= 0) & (n_l < NBODY)
        base_i = jnp.where(body_ok,
                           bsel * BODY_B + (n_l >> 7) * 512 + (n_l & 127),
                           cellv)
        for j in range(4):
            plsc.store_scatter(idxbuf, [v * 64 + iota4 + j], base_i + j * 128)
    pltpu.async_copy(body_hbm.at[idxbuf], rows, sem).wait()
    for v in range(PER_W_CELLS // 16):
        fbase = v * 64
        i0 = _round_bf16(plsc.load_gather(rows, [fbase + iota4]))
        i1 = _round_bf16(plsc.load_gather(rows, [fbase + iota4 + 1]))
        i2 = _round_bf16(plsc.load_gather(rows, [fbase + iota4 + 2]))
        i3 = _round_bf16(plsc.load_gather(rows, [fbase + iota4 + 3]))
        Zb = t[8] * i0 + t[9] * i1 + t[10] * i2 + t[11] * i3
        best = bestbuf[pl.ds(v * 16, 16)]
        bsel = (best >= NB).astype(jnp.int32)
        n_l = best - bsel * NB
        is_tail = (best >= 0) & (n_l >= NBODY)
        tix = jnp.minimum(jnp.maximum(n_l - NBODY, 0), NTAIL - 1) + bsel * 256
        j0 = _round_bf16(plsc.load_gather(tailv, [tix]))
        j1 = _round_bf16(plsc.load_gather(tailv, [tix + 64]))
        j2 = _round_bf16(plsc.load_gather(tailv, [tix + 128]))
        j3 = _round_bf16(plsc.load_gather(tailv, [tix + 192]))
        Zt = t[8] * j0 + t[9] * j1 + t[10] * j2 + t[11] * j3
        Z = jnp.where(is_tail, Zt, Zb)
        outbuf[pl.ds(v * 16, 16)] = jnp.where(best >= 0, Z, 0.0)
    pltpu.sync_copy(outbuf, out_hbm.at[pl.ds(cell0, PER_W_CELLS)])


def kernel(inputs, trans):
    body = lax.reshape(
        jnp.reshape(inputs[:, :NBODY, :], (2, NBLK, 128, 4)),
        (2 * BODY_B,), dimensions=(0, 1, 3, 2))
    tail = lax.reshape(inputs[:, NBODY:, :], (512,), dimensions=(0, 2, 1))
    tpad = jnp.zeros((16,), jnp.float32).at[:12].set(trans.ravel())
    bestn = _phase_a(body, tail, tpad)
    depth = _phase_b(body, tail, tpad, bestn)
    return depth[:CELLS].reshape(2, 37, 120)
